# baseline (speedup 1.0000x reference)
import functools

import jax
import jax.numpy as jnp
from jax.experimental import pallas as pl
from jax.experimental.pallas import tpu as pltpu

N = 10000
E = 320000
D = 128
HID = 256
NUM_EDGES = 320000
NG = 16


def _heads_kernel(pooled_ref, Wa1_ref, ba1_ref, Wa2_ref, ba2_ref,
                  Wc1_ref, bc1_ref, Wc2_ref, bc2_ref,
                  action_ref, value_ref):
    j = pl.program_id(0)
    pooled = pooled_ref[...]
    ha = jax.nn.relu(pooled @ Wa1_ref[...] + ba1_ref[...][None, :])
    action_ref[...] = jnp.tanh(ha @ Wa2_ref[...] + ba2_ref[...])

    @pl.when(j == 0)
    def _():
        hc = jax.nn.relu(pooled @ Wc1_ref[...] + bc1_ref[...][None, :])
        value_ref[...] = hc @ Wc2_ref[...] + bc2_ref[...][None, :]


def _mlp_heads(pooled, Wa1, ba1, Wa2, ba2, Wc1, bc1, Wc2, bc2):
    BLK = 16000
    grid = (NUM_EDGES // BLK,)
    action, value = pl.pallas_call(
        _heads_kernel,
        grid=grid,
        in_specs=[
            pl.BlockSpec((NG, HID), lambda j: (0, 0)),
            pl.BlockSpec((HID, HID), lambda j: (0, 0)),
            pl.BlockSpec((HID,), lambda j: (0,)),
            pl.BlockSpec((HID, BLK), lambda j: (0, j)),
            pl.BlockSpec((1, BLK), lambda j: (0, j)),
            pl.BlockSpec((HID, HID), lambda j: (0, 0)),
            pl.BlockSpec((HID,), lambda j: (0,)),
            pl.BlockSpec((HID, 1), lambda j: (0, 0)),
            pl.BlockSpec((1,), lambda j: (0,)),
        ],
        out_specs=[
            pl.BlockSpec((NG, BLK), lambda j: (0, j)),
            pl.BlockSpec((NG, 1), lambda j: (0, 0)),
        ],
        out_shape=[
            jax.ShapeDtypeStruct((NG, NUM_EDGES), jnp.float32),
            jax.ShapeDtypeStruct((NG, 1), jnp.float32),
        ],
    )(pooled, Wa1, ba1, Wa2, ba2.reshape(1, -1), Wc1, bc1, Wc2, bc2)
    return action, value


def _gat(x, src, dst, W, a_s, a_d, b, H, C, concat):
    n = x.shape[0]
    h = (x @ W).reshape(n, H, C)
    alpha_src = (h * a_s[None, :, :]).sum(-1)
    alpha_dst = (h * a_d[None, :, :]).sum(-1)
    e = alpha_src[src] + alpha_dst[dst]
    e = jax.nn.leaky_relu(e, 0.2)
    emax = jax.ops.segment_max(e, dst, num_segments=n)
    e = jnp.exp(e - emax[dst])
    denom = jax.ops.segment_sum(e, dst, num_segments=n)
    alpha = e / (denom[dst] + 1e-16)
    msg = h[src] * alpha[:, :, None]
    out = jax.ops.segment_sum(msg, dst, num_segments=n)
    if concat:
        out = out.reshape(n, H * C)
    else:
        out = out.mean(axis=1)
    return out + b


def kernel(x, edge_index, batch, W1, a1s, a1d, b1, W2, a2s, a2d, b2,
           W3, a3s, a3d, b3, Wa1, ba1, Wa2, ba2, Wc1, bc1, Wc2, bc2):
    n = x.shape[0]
    loop = jnp.arange(n, dtype=edge_index.dtype)
    src = jnp.concatenate([edge_index[0], loop])
    dst = jnp.concatenate([edge_index[1], loop])
    h = jax.nn.relu(_gat(x, src, dst, W1, a1s, a1d, b1, 4, HID, True))
    h = jax.nn.relu(_gat(h, src, dst, W2, a2s, a2d, b2, 4, HID, True))
    h = jax.nn.relu(_gat(h, src, dst, W3, a3s, a3d, b3, 1, HID, False))
    counts = jax.ops.segment_sum(jnp.ones((n,), jnp.float32), batch, num_segments=NG)
    pooled = jax.ops.segment_sum(h, batch, num_segments=NG) / jnp.maximum(counts, 1.0)[:, None]
    action, value = _mlp_heads(pooled, Wa1, ba1, Wa2, ba2, Wc1, bc1, Wc2, bc2)
    return (action, value)


# speedup vs baseline: 8.9268x; 8.9268x over previous
import functools

import jax
import jax.numpy as jnp
from jax import lax
from jax.experimental import pallas as pl
from jax.experimental.pallas import tpu as pltpu
from jax.experimental.pallas import tpu_sc as plsc

NN = 10000
EE = 320000
DD = 128
HID = 256
OUT_EDGES = 320000
NG = 16

NB = 256
NBLK = 40
NPAD = NB * NBLK
K = 512
ETOT = EE + NN
NCH = 688
EPAD = NCH * K

_pallas_call = pl.pallas_call


def _preprocess(edge_index):
    loop = jnp.arange(NN, dtype=jnp.int32)
    src = jnp.concatenate([edge_index[0].astype(jnp.int32), loop])
    dst = jnp.concatenate([edge_index[1].astype(jnp.int32), loop])
    order = jnp.argsort(dst)
    src_s = src[order]
    dst_s = dst[order]
    bstart = jnp.searchsorted(dst_s, jnp.arange(NBLK + 1, dtype=jnp.int32) * NB
                              ).astype(jnp.int32)
    cnt = bstart[1:] - bstart[:-1]
    nch_b = (cnt + K - 1) // K
    padstart = K * jnp.concatenate(
        [jnp.zeros((1,), jnp.int32), jnp.cumsum(nch_b).astype(jnp.int32)])
    blkmap = jnp.clip(
        jnp.searchsorted(padstart, jnp.arange(NCH, dtype=jnp.int32) * K,
                         side='right').astype(jnp.int32) - 1, 0, NBLK - 1)
    t = jnp.arange(EPAD, dtype=jnp.int32)
    b_t = jnp.clip(jnp.searchsorted(padstart, t, side='right'
                                    ).astype(jnp.int32) - 1, 0, NBLK - 1)
    o_t = t - padstart[b_t]
    valid = o_t < cnt[b_t]
    e_t = jnp.minimum(bstart[b_t] + jnp.minimum(o_t, jnp.maximum(cnt[b_t] - 1, 0)),
                      ETOT - 1)
    src_pad = jnp.where(valid, src_s[e_t], 0).astype(jnp.int32)
    dst_pad = jnp.where(valid, dst_s[e_t], 0).astype(jnp.int32)
    dstloc = jnp.where(valid, dst_s[e_t] - b_t * NB, 0).astype(jnp.int32)
    maskf = valid.astype(jnp.float32)
    return src_pad, dst_pad, dstloc, maskf, blkmap


def _sc_mesh():
    return plsc.VectorSubcoreMesh(core_axis_name="c", subcore_axis_name="s")


def _sc_gather_rows(table, idx, width, kw):
    nwork = 32
    b_per_w = EPAD // nwork
    nwin = b_per_w // kw

    @functools.partial(
        pl.kernel,
        out_type=jax.ShapeDtypeStruct((EPAD, width), table.dtype),
        mesh=_sc_mesh(),
        scratch_types=[
            pltpu.VMEM((b_per_w,), jnp.int32),
            pltpu.VMEM((kw, width), jnp.float32),
            pltpu.VMEM((kw, width), jnp.float32),
            pltpu.SemaphoreType.DMA,
            pltpu.SemaphoreType.DMA,
        ])
    def k(tab_hbm, idx_hbm, o_hbm, idx_v, rows0, rows1, sem0, sem1):
        wid = lax.axis_index("s") * 2 + lax.axis_index("c")
        base = wid * b_per_w
        pltpu.sync_copy(idx_hbm.at[pl.ds(base, b_per_w)], idx_v)
        rows = (rows0, rows1)
        sems = (sem0, sem1)

        def issue(w, b):
            pltpu.async_copy(tab_hbm.at[idx_v.at[pl.ds(w * kw, kw)]],
                             rows[b], sems[b])

        issue(0, 0)
        issue(1, 1)

        @pl.loop(0, nwin, step=2)
        def _(w):
            for par in range(2):
                ww = w + par
                pltpu.make_async_copy(
                    tab_hbm.at[idx_v.at[pl.ds(ww * kw, kw)]],
                    rows[par], sems[par]).wait()
                pltpu.sync_copy(rows[par],
                                o_hbm.at[pl.ds(base + ww * kw, kw)])
                nxt = ww + 2

                @pl.when(nxt < nwin)
                def _():
                    issue(nxt, par)

    return k(table, idx)


def _dense_kernel(act_ref, w_ref, ad_ref, h_ref, atd_ref):
    hb = jnp.dot(act_ref[...], w_ref[...], preferred_element_type=jnp.float32)
    h_ref[...] = hb
    atd_ref[...] = jnp.dot(hb, ad_ref[...], preferred_element_type=jnp.float32)


def _att_matrix(a, nheads, C, dout):
    m = jnp.zeros((dout, 16), jnp.float32)
    for h in range(nheads):
        m = m.at[h * C:(h + 1) * C, h].set(a[h])
    return m


def _dense(act, W, a_d, nheads, C):
    din, dout = W.shape
    a_mat_d = _att_matrix(a_d, nheads, C, dout)
    return _pallas_call(
        _dense_kernel,
        grid=(NPAD // NB,),
        in_specs=[
            pl.BlockSpec((NB, din), lambda j: (j, 0)),
            pl.BlockSpec((din, dout), lambda j: (0, 0)),
            pl.BlockSpec((dout, 16), lambda j: (0, 0)),
        ],
        out_specs=[
            pl.BlockSpec((NB, dout), lambda j: (j, 0)),
            pl.BlockSpec((NB, 16), lambda j: (j, 0)),
        ],
        out_shape=[
            jax.ShapeDtypeStruct((NPAD, dout), jnp.float32),
            jax.ShapeDtypeStruct((NPAD, 16), jnp.float32),
        ],
    )(act, W, a_mat_d)


def _agg_body(nheads, C, blkmap_ref, msg_ref, as_ref, atd_ref, dl_ref, mk_ref,
              out_ref, den_ref):
    c = pl.program_id(0)
    blk = blkmap_ref[c]
    prev = blkmap_ref[jnp.maximum(c - 1, 0)]
    first = jnp.logical_or(c == 0, blk != prev)
    msg = msg_ref[...]
    dl = dl_ref[0, 0, :]
    onehot = (dl[:, None] == lax.broadcasted_iota(jnp.int32, (K, NB), 1)
              ).astype(jnp.float32)
    es = jnp.dot(msg, as_ref[...], preferred_element_type=jnp.float32)
    ed = jnp.dot(onehot, atd_ref[...], preferred_element_type=jnp.float32)
    asum = es + ed
    e = jnp.maximum(asum, 0.2 * asum)
    mask = mk_ref[0, 0, :]
    p = jnp.exp(e) * mask[:, None]
    den_c = lax.dot_general(onehot, p, (((0,), (0,)), ((), ())),
                            preferred_element_type=jnp.float32)
    outs = []
    for h in range(nheads):
        ph = onehot * p[:, h][:, None]
        outs.append(lax.dot_general(ph, msg[:, h * C:(h + 1) * C],
                                    (((0,), (0,)), ((), ())),
                                    preferred_element_type=jnp.float32))
    out_c = jnp.concatenate(outs, axis=1) if nheads > 1 else outs[0]

    @pl.when(first)
    def _():
        out_ref[...] = out_c
        den_ref[...] = den_c

    @pl.when(jnp.logical_not(first))
    def _():
        out_ref[...] = out_ref[...] + out_c
        den_ref[...] = den_ref[...] + den_c


def _aggregate(msg, a_s, atd, dstloc, maskf, blkmap, nheads, C):
    wd = nheads * C
    a_mat_s = _att_matrix(a_s, nheads, C, wd)
    dl3 = dstloc.reshape(NCH, 1, K)
    mk3 = maskf.reshape(NCH, 1, K)
    grid_spec = pltpu.PrefetchScalarGridSpec(
        num_scalar_prefetch=1,
        grid=(NCH,),
        in_specs=[
            pl.BlockSpec((K, wd), lambda c, bm: (c, 0)),
            pl.BlockSpec((wd, 16), lambda c, bm: (0, 0)),
            pl.BlockSpec((NB, 16), lambda c, bm: (bm[c], 0)),
            pl.BlockSpec((1, 1, K), lambda c, bm: (c, 0, 0)),
            pl.BlockSpec((1, 1, K), lambda c, bm: (c, 0, 0)),
        ],
        out_specs=[
            pl.BlockSpec((NB, wd), lambda c, bm: (bm[c], 0)),
            pl.BlockSpec((NB, 16), lambda c, bm: (bm[c], 0)),
        ],
    )
    return _pallas_call(
        functools.partial(_agg_body, nheads, C),
        grid_spec=grid_spec,
        out_shape=[
            jax.ShapeDtypeStruct((NPAD, wd), jnp.float32),
            jax.ShapeDtypeStruct((NPAD, 16), jnp.float32),
        ],
        compiler_params=pltpu.CompilerParams(
            dimension_semantics=("arbitrary",)),
    )(blkmap, msg, a_mat_s, atd, dl3, mk3)


def _norm_body(nheads, C, mean_heads, out_ref, den_ref, b_ref, act_ref):
    den = den_ref[...]
    parts = []
    for h in range(nheads):
        s = den[:, h:h + 1]
        s_safe = jnp.where(s > 0, s, 1.0)
        parts.append(out_ref[:, h * C:(h + 1) * C] / s_safe)
    v = jnp.concatenate(parts, axis=1) if nheads > 1 else parts[0]
    if mean_heads and nheads > 1:
        v = sum(parts) / nheads
    act_ref[...] = jnp.maximum(v + b_ref[...], 0.0)


def _normalize(out, den, bias, nheads, C, mean_heads=False):
    wd = C if (mean_heads or nheads == 1) else nheads * C
    win = nheads * C
    return _pallas_call(
        functools.partial(_norm_body, nheads, C, mean_heads),
        grid=(NPAD // NB,),
        in_specs=[
            pl.BlockSpec((NB, win), lambda j: (j, 0)),
            pl.BlockSpec((NB, 16), lambda j: (j, 0)),
            pl.BlockSpec((1, wd), lambda j: (0, 0)),
        ],
        out_specs=pl.BlockSpec((NB, wd), lambda j: (j, 0)),
        out_shape=jax.ShapeDtypeStruct((NPAD, wd), jnp.float32),
    )(out, den, bias.reshape(1, wd))


def _pool_body(h_ref, b_ref, psum_ref, pcnt_ref):
    j = pl.program_id(0)
    bt = b_ref[0, 0, :]
    oh = (bt[:, None] == lax.broadcasted_iota(jnp.int32, (NB, NG), 1)
          ).astype(jnp.float32)
    ps = lax.dot_general(oh, h_ref[...], (((0,), (0,)), ((), ())),
                         preferred_element_type=jnp.float32)
    pc = lax.dot_general(oh, jnp.ones((NB, 16), jnp.float32),
                         (((0,), (0,)), ((), ())),
                         preferred_element_type=jnp.float32)

    @pl.when(j == 0)
    def _():
        psum_ref[...] = ps
        pcnt_ref[...] = pc

    @pl.when(j != 0)
    def _():
        psum_ref[...] = psum_ref[...] + ps
        pcnt_ref[...] = pcnt_ref[...] + pc


def _pool(h, batch_pad):
    b3 = batch_pad.reshape(NBLK, 1, NB)
    return _pallas_call(
        _pool_body,
        grid=(NBLK,),
        in_specs=[
            pl.BlockSpec((NB, HID), lambda j: (j, 0)),
            pl.BlockSpec((1, 1, NB), lambda j: (j, 0, 0)),
        ],
        out_specs=[
            pl.BlockSpec((NG, HID), lambda j: (0, 0)),
            pl.BlockSpec((NG, 16), lambda j: (0, 0)),
        ],
        out_shape=[
            jax.ShapeDtypeStruct((NG, HID), jnp.float32),
            jax.ShapeDtypeStruct((NG, 16), jnp.float32),
        ],
    )(h, b3)


def _heads_kernel(psum_ref, pcnt_ref, Wa1_ref, ba1_ref, Wa2_ref, ba2_ref,
                  Wc1_ref, bc1_ref, Wc2_ref, bc2_ref,
                  action_ref, value_ref):
    j = pl.program_id(0)
    cnt = jnp.maximum(pcnt_ref[:, 0:1], 1.0)
    pooled = psum_ref[...] / cnt
    ha = jax.nn.relu(pooled @ Wa1_ref[...] + ba1_ref[...][None, :])
    action_ref[...] = jnp.tanh(ha @ Wa2_ref[...] + ba2_ref[...])

    @pl.when(j == 0)
    def _():
        hc = jax.nn.relu(pooled @ Wc1_ref[...] + bc1_ref[...][None, :])
        value_ref[...] = hc @ Wc2_ref[...] + bc2_ref[...][None, :]


def _mlp_heads(psum, pcnt, Wa1, ba1, Wa2, ba2, Wc1, bc1, Wc2, bc2):
    BLK = 16000
    grid = (OUT_EDGES // BLK,)
    action, value = _pallas_call(
        _heads_kernel,
        grid=grid,
        in_specs=[
            pl.BlockSpec((NG, HID), lambda j: (0, 0)),
            pl.BlockSpec((NG, 16), lambda j: (0, 0)),
            pl.BlockSpec((HID, HID), lambda j: (0, 0)),
            pl.BlockSpec((HID,), lambda j: (0,)),
            pl.BlockSpec((HID, BLK), lambda j: (0, j)),
            pl.BlockSpec((1, BLK), lambda j: (0, j)),
            pl.BlockSpec((HID, HID), lambda j: (0, 0)),
            pl.BlockSpec((HID,), lambda j: (0,)),
            pl.BlockSpec((HID, 1), lambda j: (0, 0)),
            pl.BlockSpec((1,), lambda j: (0,)),
        ],
        out_specs=[
            pl.BlockSpec((NG, BLK), lambda j: (0, j)),
            pl.BlockSpec((NG, 1), lambda j: (0, 0)),
        ],
        out_shape=[
            jax.ShapeDtypeStruct((NG, OUT_EDGES), jnp.float32),
            jax.ShapeDtypeStruct((NG, 1), jnp.float32),
        ],
    )(psum, pcnt, Wa1, ba1, Wa2, ba2.reshape(1, -1), Wc1, bc1, Wc2, bc2)
    return action, value


def _gat_layer(act, W, a_s, a_d, b, nheads, C, idxs, mean_heads=False):
    src_pad, dst_pad, dstloc, maskf, blkmap = idxs
    h, atd = _dense(act, W, a_d, nheads, C)
    kw = 32 if nheads * C > 512 else 128
    msg = _sc_gather_rows(h, src_pad, nheads * C, kw)
    out, den = _aggregate(msg, a_s, atd, dstloc, maskf, blkmap, nheads, C)
    return _normalize(out, den, b, nheads, C, mean_heads=mean_heads)


def kernel(x, edge_index, batch, W1, a1s, a1d, b1, W2, a2s, a2d, b2,
           W3, a3s, a3d, b3, Wa1, ba1, Wa2, ba2, Wc1, bc1, Wc2, bc2):
    idxs = _preprocess(edge_index)
    x_pad = jnp.pad(x, ((0, NPAD - NN), (0, 0)))
    batch_pad = jnp.pad(batch.astype(jnp.int32), (0, NPAD - NN),
                        constant_values=NG)
    h = _gat_layer(x_pad, W1, a1s, a1d, b1, 4, HID, idxs)
    h = _gat_layer(h, W2, a2s, a2d, b2, 4, HID, idxs)
    h = _gat_layer(h, W3, a3s, a3d, b3, 1, HID, idxs, mean_heads=True)
    psum, pcnt = _pool(h, batch_pad)
    action, value = _mlp_heads(psum, pcnt, Wa1, ba1, Wa2, ba2,
                               Wc1, bc1, Wc2, bc2)
    return (action, value)


# speedup vs baseline: 10.5318x; 1.1798x over previous
import functools

import jax
import jax.numpy as jnp
from jax import lax
from jax.experimental import pallas as pl
from jax.experimental.pallas import tpu as pltpu
from jax.experimental.pallas import tpu_sc as plsc

NN = 10000
EE = 320000
DD = 128
HID = 256
OUT_EDGES = 320000
NG = 16

NB = 256
NBLK = 40
NPAD = NB * NBLK
K = 512
ETOT = EE + NN
NCH = 688
EPAD = NCH * K

_pallas_call = pl.pallas_call


def _preprocess(edge_index):
    loop = jnp.arange(NN, dtype=jnp.int32)
    src = jnp.concatenate([edge_index[0].astype(jnp.int32), loop])
    dst = jnp.concatenate([edge_index[1].astype(jnp.int32), loop])
    order = jnp.argsort(dst)
    src_s = src[order]
    dst_s = dst[order]
    bstart = jnp.searchsorted(dst_s, jnp.arange(NBLK + 1, dtype=jnp.int32) * NB
                              ).astype(jnp.int32)
    cnt = bstart[1:] - bstart[:-1]
    nch_b = (cnt + K - 1) // K
    padstart = K * jnp.concatenate(
        [jnp.zeros((1,), jnp.int32), jnp.cumsum(nch_b).astype(jnp.int32)])
    blkmap = jnp.clip(
        jnp.searchsorted(padstart, jnp.arange(NCH, dtype=jnp.int32) * K,
                         side='right').astype(jnp.int32) - 1, 0, NBLK - 1)
    t = jnp.arange(EPAD, dtype=jnp.int32)
    b_t = jnp.clip(jnp.searchsorted(padstart, t, side='right'
                                    ).astype(jnp.int32) - 1, 0, NBLK - 1)
    o_t = t - padstart[b_t]
    valid = o_t < cnt[b_t]
    e_t = jnp.minimum(bstart[b_t] + jnp.minimum(o_t, jnp.maximum(cnt[b_t] - 1, 0)),
                      ETOT - 1)
    src_pad = jnp.where(valid, src_s[e_t], 0).astype(jnp.int32)
    dst_pad = jnp.where(valid, dst_s[e_t], 0).astype(jnp.int32)
    dstloc = jnp.where(valid, dst_s[e_t] - b_t * NB, 0).astype(jnp.int32)
    maskf = valid.astype(jnp.float32)
    return src_pad, dst_pad, dstloc, maskf, blkmap


def _sc_mesh():
    return plsc.VectorSubcoreMesh(core_axis_name="c", subcore_axis_name="s")


def _sc_gather_rows(table, idx, width, kw):
    nwork = 32
    b_per_w = EPAD // nwork
    nwin = b_per_w // kw

    @functools.partial(
        pl.kernel,
        out_type=jax.ShapeDtypeStruct((EPAD, width), table.dtype),
        mesh=_sc_mesh(),
        scratch_types=[
            pltpu.VMEM((b_per_w,), jnp.int32),
            pltpu.VMEM((kw, width), table.dtype),
            pltpu.VMEM((kw, width), table.dtype),
            pltpu.SemaphoreType.DMA,
            pltpu.SemaphoreType.DMA,
        ])
    def k(tab_hbm, idx_hbm, o_hbm, idx_v, rows0, rows1, sem0, sem1):
        wid = lax.axis_index("s") * 2 + lax.axis_index("c")
        base = wid * b_per_w
        pltpu.sync_copy(idx_hbm.at[pl.ds(base, b_per_w)], idx_v)
        rows = (rows0, rows1)
        sems = (sem0, sem1)

        def issue(w, b):
            pltpu.async_copy(tab_hbm.at[idx_v.at[pl.ds(w * kw, kw)]],
                             rows[b], sems[b])

        issue(0, 0)
        issue(1, 1)

        @pl.loop(0, nwin, step=2)
        def _(w):
            for par in range(2):
                ww = w + par
                pltpu.make_async_copy(
                    tab_hbm.at[idx_v.at[pl.ds(ww * kw, kw)]],
                    rows[par], sems[par]).wait()
                pltpu.sync_copy(rows[par],
                                o_hbm.at[pl.ds(base + ww * kw, kw)])
                nxt = ww + 2

                @pl.when(nxt < nwin)
                def _():
                    issue(nxt, par)

    return k(table, idx)


def _rne_bf16_bits(x):
    u = lax.bitcast_convert_type(x, jnp.uint32)
    return (u + jnp.uint32(0x7FFF) + ((u >> 16) & jnp.uint32(1))) >> 16


def _dense_kernel(act_ref, w_ref, ad_ref, h_ref, atd_ref):
    hb = jnp.dot(act_ref[...], w_ref[...], preferred_element_type=jnp.float32)
    half = hb.shape[1] // 2
    word = _rne_bf16_bits(hb[:, :half]) | (_rne_bf16_bits(hb[:, half:]) << 16)
    h_ref[...] = lax.bitcast_convert_type(word, jnp.float32)
    atd_ref[...] = jnp.dot(hb, ad_ref[...], preferred_element_type=jnp.float32)


def _att_matrix(a, nheads, C, dout):
    m = jnp.zeros((dout, 16), jnp.float32)
    for h in range(nheads):
        m = m.at[h * C:(h + 1) * C, h].set(a[h])
    return m


def _dense(act, W, a_d, nheads, C):
    din, dout = W.shape
    a_mat_d = _att_matrix(a_d, nheads, C, dout)
    return _pallas_call(
        _dense_kernel,
        grid=(NPAD // NB,),
        in_specs=[
            pl.BlockSpec((NB, din), lambda j: (j, 0)),
            pl.BlockSpec((din, dout), lambda j: (0, 0)),
            pl.BlockSpec((dout, 16), lambda j: (0, 0)),
        ],
        out_specs=[
            pl.BlockSpec((NB, dout // 2), lambda j: (j, 0)),
            pl.BlockSpec((NB, 16), lambda j: (j, 0)),
        ],
        out_shape=[
            jax.ShapeDtypeStruct((NPAD, dout // 2), jnp.float32),
            jax.ShapeDtypeStruct((NPAD, 16), jnp.float32),
        ],
    )(act, W, a_mat_d)


def _agg_body(nheads, C, blkmap_ref, msg_ref, as_ref, atd_ref, dl_ref, mk_ref,
              out_ref, den_ref):
    c = pl.program_id(0)
    blk = blkmap_ref[c]
    prev = blkmap_ref[jnp.maximum(c - 1, 0)]
    first = jnp.logical_or(c == 0, blk != prev)
    u = lax.bitcast_convert_type(msg_ref[...], jnp.uint32)
    lo = lax.bitcast_convert_type(u << 16, jnp.float32)
    hi = lax.bitcast_convert_type(u & jnp.uint32(0xFFFF0000), jnp.float32)
    msg = jnp.concatenate([lo, hi], axis=1).astype(jnp.bfloat16)
    dl = dl_ref[0, 0, :]
    onehot = (dl[:, None] == lax.broadcasted_iota(jnp.int32, (K, NB), 1)
              ).astype(jnp.float32)
    es = jnp.dot(msg, as_ref[...].astype(jnp.bfloat16),
                 preferred_element_type=jnp.float32)
    ed = jnp.dot(onehot, atd_ref[...], preferred_element_type=jnp.float32)
    asum = es + ed
    e = jnp.maximum(asum, 0.2 * asum)
    mask = mk_ref[0, 0, :]
    p = jnp.exp(e) * mask[:, None]
    den_c = lax.dot_general(onehot, p, (((0,), (0,)), ((), ())),
                            preferred_element_type=jnp.float32)
    outs = []
    for h in range(nheads):
        ph = (onehot * p[:, h][:, None]).astype(jnp.bfloat16)
        outs.append(lax.dot_general(ph, msg[:, h * C:(h + 1) * C],
                                    (((0,), (0,)), ((), ())),
                                    preferred_element_type=jnp.float32))
    out_c = jnp.concatenate(outs, axis=1) if nheads > 1 else outs[0]

    @pl.when(first)
    def _():
        out_ref[...] = out_c
        den_ref[...] = den_c

    @pl.when(jnp.logical_not(first))
    def _():
        out_ref[...] = out_ref[...] + out_c
        den_ref[...] = den_ref[...] + den_c


def _aggregate(msg, a_s, atd, dstloc, maskf, blkmap, nheads, C):
    wd = nheads * C
    a_mat_s = _att_matrix(a_s, nheads, C, wd)
    dl3 = dstloc.reshape(NCH, 1, K)
    mk3 = maskf.reshape(NCH, 1, K)
    grid_spec = pltpu.PrefetchScalarGridSpec(
        num_scalar_prefetch=1,
        grid=(NCH,),
        in_specs=[
            pl.BlockSpec((K, wd // 2), lambda c, bm: (c, 0)),
            pl.BlockSpec((wd, 16), lambda c, bm: (0, 0)),
            pl.BlockSpec((NB, 16), lambda c, bm: (bm[c], 0)),
            pl.BlockSpec((1, 1, K), lambda c, bm: (c, 0, 0)),
            pl.BlockSpec((1, 1, K), lambda c, bm: (c, 0, 0)),
        ],
        out_specs=[
            pl.BlockSpec((NB, wd), lambda c, bm: (bm[c], 0)),
            pl.BlockSpec((NB, 16), lambda c, bm: (bm[c], 0)),
        ],
    )
    return _pallas_call(
        functools.partial(_agg_body, nheads, C),
        grid_spec=grid_spec,
        out_shape=[
            jax.ShapeDtypeStruct((NPAD, wd), jnp.float32),
            jax.ShapeDtypeStruct((NPAD, 16), jnp.float32),
        ],
        compiler_params=pltpu.CompilerParams(
            dimension_semantics=("arbitrary",)),
    )(blkmap, msg, a_mat_s, atd, dl3, mk3)


def _norm_body(nheads, C, mean_heads, out_ref, den_ref, b_ref, act_ref):
    den = den_ref[...]
    parts = []
    for h in range(nheads):
        s = den[:, h:h + 1]
        s_safe = jnp.where(s > 0, s, 1.0)
        parts.append(out_ref[:, h * C:(h + 1) * C] / s_safe)
    v = jnp.concatenate(parts, axis=1) if nheads > 1 else parts[0]
    if mean_heads and nheads > 1:
        v = sum(parts) / nheads
    act_ref[...] = jnp.maximum(v + b_ref[...], 0.0)


def _normalize(out, den, bias, nheads, C, mean_heads=False):
    wd = C if (mean_heads or nheads == 1) else nheads * C
    win = nheads * C
    return _pallas_call(
        functools.partial(_norm_body, nheads, C, mean_heads),
        grid=(NPAD // NB,),
        in_specs=[
            pl.BlockSpec((NB, win), lambda j: (j, 0)),
            pl.BlockSpec((NB, 16), lambda j: (j, 0)),
            pl.BlockSpec((1, wd), lambda j: (0, 0)),
        ],
        out_specs=pl.BlockSpec((NB, wd), lambda j: (j, 0)),
        out_shape=jax.ShapeDtypeStruct((NPAD, wd), jnp.float32),
    )(out, den, bias.reshape(1, wd))


def _pool_body(h_ref, b_ref, psum_ref, pcnt_ref):
    j = pl.program_id(0)
    bt = b_ref[0, 0, :]
    oh = (bt[:, None] == lax.broadcasted_iota(jnp.int32, (NB, NG), 1)
          ).astype(jnp.float32)
    ps = lax.dot_general(oh, h_ref[...], (((0,), (0,)), ((), ())),
                         preferred_element_type=jnp.float32)
    pc = lax.dot_general(oh, jnp.ones((NB, 16), jnp.float32),
                         (((0,), (0,)), ((), ())),
                         preferred_element_type=jnp.float32)

    @pl.when(j == 0)
    def _():
        psum_ref[...] = ps
        pcnt_ref[...] = pc

    @pl.when(j != 0)
    def _():
        psum_ref[...] = psum_ref[...] + ps
        pcnt_ref[...] = pcnt_ref[...] + pc


def _pool(h, batch_pad):
    b3 = batch_pad.reshape(NBLK, 1, NB)
    return _pallas_call(
        _pool_body,
        grid=(NBLK,),
        in_specs=[
            pl.BlockSpec((NB, HID), lambda j: (j, 0)),
            pl.BlockSpec((1, 1, NB), lambda j: (j, 0, 0)),
        ],
        out_specs=[
            pl.BlockSpec((NG, HID), lambda j: (0, 0)),
            pl.BlockSpec((NG, 16), lambda j: (0, 0)),
        ],
        out_shape=[
            jax.ShapeDtypeStruct((NG, HID), jnp.float32),
            jax.ShapeDtypeStruct((NG, 16), jnp.float32),
        ],
    )(h, b3)


def _heads_kernel(psum_ref, pcnt_ref, Wa1_ref, ba1_ref, Wa2_ref, ba2_ref,
                  Wc1_ref, bc1_ref, Wc2_ref, bc2_ref,
                  action_ref, value_ref):
    j = pl.program_id(0)
    cnt = jnp.maximum(pcnt_ref[:, 0:1], 1.0)
    pooled = psum_ref[...] / cnt
    ha = jax.nn.relu(pooled @ Wa1_ref[...] + ba1_ref[...][None, :])
    action_ref[...] = jnp.tanh(ha @ Wa2_ref[...] + ba2_ref[...])

    @pl.when(j == 0)
    def _():
        hc = jax.nn.relu(pooled @ Wc1_ref[...] + bc1_ref[...][None, :])
        value_ref[...] = hc @ Wc2_ref[...] + bc2_ref[...][None, :]


def _mlp_heads(psum, pcnt, Wa1, ba1, Wa2, ba2, Wc1, bc1, Wc2, bc2):
    BLK = 16000
    grid = (OUT_EDGES // BLK,)
    action, value = _pallas_call(
        _heads_kernel,
        grid=grid,
        in_specs=[
            pl.BlockSpec((NG, HID), lambda j: (0, 0)),
            pl.BlockSpec((NG, 16), lambda j: (0, 0)),
            pl.BlockSpec((HID, HID), lambda j: (0, 0)),
            pl.BlockSpec((HID,), lambda j: (0,)),
            pl.BlockSpec((HID, BLK), lambda j: (0, j)),
            pl.BlockSpec((1, BLK), lambda j: (0, j)),
            pl.BlockSpec((HID, HID), lambda j: (0, 0)),
            pl.BlockSpec((HID,), lambda j: (0,)),
            pl.BlockSpec((HID, 1), lambda j: (0, 0)),
            pl.BlockSpec((1,), lambda j: (0,)),
        ],
        out_specs=[
            pl.BlockSpec((NG, BLK), lambda j: (0, j)),
            pl.BlockSpec((NG, 1), lambda j: (0, 0)),
        ],
        out_shape=[
            jax.ShapeDtypeStruct((NG, OUT_EDGES), jnp.float32),
            jax.ShapeDtypeStruct((NG, 1), jnp.float32),
        ],
    )(psum, pcnt, Wa1, ba1, Wa2, ba2.reshape(1, -1), Wc1, bc1, Wc2, bc2)
    return action, value


def _gat_layer(act, W, a_s, a_d, b, nheads, C, idxs, mean_heads=False):
    src_pad, dst_pad, dstloc, maskf, blkmap = idxs
    h, atd = _dense(act, W, a_d, nheads, C)
    kw = 64 if nheads * C > 512 else 128
    msg = _sc_gather_rows(h, src_pad, nheads * C // 2, kw)
    out, den = _aggregate(msg, a_s, atd, dstloc, maskf, blkmap, nheads, C)
    return _normalize(out, den, b, nheads, C, mean_heads=mean_heads)


def kernel(x, edge_index, batch, W1, a1s, a1d, b1, W2, a2s, a2d, b2,
           W3, a3s, a3d, b3, Wa1, ba1, Wa2, ba2, Wc1, bc1, Wc2, bc2):
    idxs = _preprocess(edge_index)
    x_pad = jnp.pad(x, ((0, NPAD - NN), (0, 0)))
    batch_pad = jnp.pad(batch.astype(jnp.int32), (0, NPAD - NN),
                        constant_values=NG)
    h = _gat_layer(x_pad, W1, a1s, a1d, b1, 4, HID, idxs)
    h = _gat_layer(h, W2, a2s, a2d, b2, 4, HID, idxs)
    h = _gat_layer(h, W3, a3s, a3d, b3, 1, HID, idxs, mean_heads=True)
    psum, pcnt = _pool(h, batch_pad)
    action, value = _mlp_heads(psum, pcnt, Wa1, ba1, Wa2, ba2,
                               Wc1, bc1, Wc2, bc2)
    return (action, value)


# speedup vs baseline: 10.9325x; 1.0380x over previous
import functools

import jax
import jax.numpy as jnp
from jax import lax
from jax.experimental import pallas as pl
from jax.experimental.pallas import tpu as pltpu
from jax.experimental.pallas import tpu_sc as plsc

NN = 10000
EE = 320000
DD = 128
HID = 256
OUT_EDGES = 320000
NG = 16

NB = 256
NBLK = 40
NPAD = NB * NBLK
K = 512
ETOT = EE + NN
NCH = 688
EPAD = NCH * K

_pallas_call = pl.pallas_call


def _preprocess(edge_index):
    loop = jnp.arange(NN, dtype=jnp.int32)
    src = jnp.concatenate([edge_index[0].astype(jnp.int32), loop])
    dst = jnp.concatenate([edge_index[1].astype(jnp.int32), loop])
    order = jnp.argsort(dst)
    src_s = src[order]
    dst_s = dst[order]
    bstart = jnp.searchsorted(dst_s, jnp.arange(NBLK + 1, dtype=jnp.int32) * NB
                              ).astype(jnp.int32)
    cnt = bstart[1:] - bstart[:-1]
    nch_b = (cnt + K - 1) // K
    padstart = K * jnp.concatenate(
        [jnp.zeros((1,), jnp.int32), jnp.cumsum(nch_b).astype(jnp.int32)])
    blkmap = jnp.clip(
        jnp.searchsorted(padstart, jnp.arange(NCH, dtype=jnp.int32) * K,
                         side='right').astype(jnp.int32) - 1, 0, NBLK - 1)
    t = jnp.arange(EPAD, dtype=jnp.int32)
    b_t = jnp.clip(jnp.searchsorted(padstart, t, side='right'
                                    ).astype(jnp.int32) - 1, 0, NBLK - 1)
    o_t = t - padstart[b_t]
    valid = o_t < cnt[b_t]
    e_t = jnp.minimum(bstart[b_t] + jnp.minimum(o_t, jnp.maximum(cnt[b_t] - 1, 0)),
                      ETOT - 1)
    src_pad = jnp.where(valid, src_s[e_t], 0).astype(jnp.int32)
    dst_pad = jnp.where(valid, dst_s[e_t], 0).astype(jnp.int32)
    dstloc = jnp.where(valid, dst_s[e_t] - b_t * NB, 0).astype(jnp.int32)
    maskf = valid.astype(jnp.float32)
    return src_pad, dst_pad, dstloc, maskf, blkmap


def _sc_mesh():
    return plsc.VectorSubcoreMesh(core_axis_name="c", subcore_axis_name="s")


def _sc_gather_rows(table, idx, width, kw):
    nwork = 32
    b_per_w = EPAD // nwork
    nwin = b_per_w // kw

    @functools.partial(
        pl.kernel,
        out_type=jax.ShapeDtypeStruct((EPAD, width), table.dtype),
        mesh=_sc_mesh(),
        scratch_types=[
            pltpu.VMEM((b_per_w,), jnp.int32),
            pltpu.VMEM((kw, width), table.dtype),
            pltpu.VMEM((kw, width), table.dtype),
            pltpu.SemaphoreType.DMA,
            pltpu.SemaphoreType.DMA,
        ])
    def k(tab_hbm, idx_hbm, o_hbm, idx_v, rows0, rows1, sem0, sem1):
        wid = lax.axis_index("s") * 2 + lax.axis_index("c")
        base = wid * b_per_w
        pltpu.sync_copy(idx_hbm.at[pl.ds(base, b_per_w)], idx_v)
        rows = (rows0, rows1)
        sems = (sem0, sem1)

        def issue(w, b):
            pltpu.async_copy(tab_hbm.at[idx_v.at[pl.ds(w * kw, kw)]],
                             rows[b], sems[b])

        issue(0, 0)
        issue(1, 1)

        @pl.loop(0, nwin, step=2)
        def _(w):
            for par in range(2):
                ww = w + par
                pltpu.make_async_copy(
                    tab_hbm.at[idx_v.at[pl.ds(ww * kw, kw)]],
                    rows[par], sems[par]).wait()
                pltpu.sync_copy(rows[par],
                                o_hbm.at[pl.ds(base + ww * kw, kw)])
                nxt = ww + 2

                @pl.when(nxt < nwin)
                def _():
                    issue(nxt, par)

    return k(table, idx)


def _rne_bf16_bits(x):
    u = lax.bitcast_convert_type(x, jnp.uint32)
    return (u + jnp.uint32(0x7FFF) + ((u >> 16) & jnp.uint32(1))) >> 16


def _dense_kernel(act_ref, w_ref, ad_ref, h_ref, atd_ref):
    hb = jnp.dot(act_ref[...], w_ref[...], preferred_element_type=jnp.float32)
    half = hb.shape[1] // 2
    word = _rne_bf16_bits(hb[:, :half]) | (_rne_bf16_bits(hb[:, half:]) << 16)
    h_ref[...] = lax.bitcast_convert_type(word, jnp.float32)
    atd_ref[...] = jnp.dot(hb, ad_ref[...], preferred_element_type=jnp.float32)


def _att_matrix(a, nheads, C, dout):
    m = jnp.zeros((dout, 16), jnp.float32)
    for h in range(nheads):
        m = m.at[h * C:(h + 1) * C, h].set(a[h])
    return m


def _dense(act, W, a_d, nheads, C):
    din, dout = W.shape
    a_mat_d = _att_matrix(a_d, nheads, C, dout)
    return _pallas_call(
        _dense_kernel,
        grid=(NPAD // NB,),
        in_specs=[
            pl.BlockSpec((NB, din), lambda j: (j, 0)),
            pl.BlockSpec((din, dout), lambda j: (0, 0)),
            pl.BlockSpec((dout, 16), lambda j: (0, 0)),
        ],
        out_specs=[
            pl.BlockSpec((NB, dout // 2), lambda j: (j, 0)),
            pl.BlockSpec((NB, 16), lambda j: (j, 0)),
        ],
        out_shape=[
            jax.ShapeDtypeStruct((NPAD, dout // 2), jnp.float32),
            jax.ShapeDtypeStruct((NPAD, 16), jnp.float32),
        ],
    )(act, W, a_mat_d)


def _agg_body(nheads, C, blkmap_ref, msg_ref, as_ref, atd_ref, dl_ref, mk_ref,
              out_ref, den_ref):
    c = pl.program_id(0)
    blk = blkmap_ref[c]
    prev = blkmap_ref[jnp.maximum(c - 1, 0)]
    first = jnp.logical_or(c == 0, blk != prev)
    u = lax.bitcast_convert_type(msg_ref[...], jnp.uint32)
    lo = lax.bitcast_convert_type(u << 16, jnp.float32)
    hi = lax.bitcast_convert_type(u & jnp.uint32(0xFFFF0000), jnp.float32)
    msg = jnp.concatenate([lo, hi], axis=1).astype(jnp.bfloat16)
    dl = dl_ref[0, 0, :]
    onehot = (dl[:, None] == lax.broadcasted_iota(jnp.int32, (K, NB), 1)
              ).astype(jnp.float32)
    onehot_t = (lax.broadcasted_iota(jnp.int32, (NB, K), 0) == dl[None, :])
    oht_b = onehot_t.astype(jnp.bfloat16)
    es = jnp.dot(msg, as_ref[...].astype(jnp.bfloat16),
                 preferred_element_type=jnp.float32)
    ed = jnp.dot(onehot, atd_ref[...], preferred_element_type=jnp.float32)
    asum = es + ed
    e = jnp.maximum(asum, 0.2 * asum)
    mask = mk_ref[0, 0, :]
    p = jnp.exp(e) * mask[:, None]
    den_c = jnp.dot(onehot_t.astype(jnp.float32), p,
                    preferred_element_type=jnp.float32)
    pb = p.astype(jnp.bfloat16)
    outs = []
    for h in range(nheads):
        ph_t = oht_b * pb[:, h][None, :]
        outs.append(jnp.dot(ph_t, msg[:, h * C:(h + 1) * C],
                            preferred_element_type=jnp.float32))
    out_c = jnp.concatenate(outs, axis=1) if nheads > 1 else outs[0]

    @pl.when(first)
    def _():
        out_ref[...] = out_c
        den_ref[...] = den_c

    @pl.when(jnp.logical_not(first))
    def _():
        out_ref[...] = out_ref[...] + out_c
        den_ref[...] = den_ref[...] + den_c


def _aggregate(msg, a_s, atd, dstloc, maskf, blkmap, nheads, C):
    wd = nheads * C
    a_mat_s = _att_matrix(a_s, nheads, C, wd)
    dl3 = dstloc.reshape(NCH, 1, K)
    mk3 = maskf.reshape(NCH, 1, K)
    grid_spec = pltpu.PrefetchScalarGridSpec(
        num_scalar_prefetch=1,
        grid=(NCH,),
        in_specs=[
            pl.BlockSpec((K, wd // 2), lambda c, bm: (c, 0)),
            pl.BlockSpec((wd, 16), lambda c, bm: (0, 0)),
            pl.BlockSpec((NB, 16), lambda c, bm: (bm[c], 0)),
            pl.BlockSpec((1, 1, K), lambda c, bm: (c, 0, 0)),
            pl.BlockSpec((1, 1, K), lambda c, bm: (c, 0, 0)),
        ],
        out_specs=[
            pl.BlockSpec((NB, wd), lambda c, bm: (bm[c], 0)),
            pl.BlockSpec((NB, 16), lambda c, bm: (bm[c], 0)),
        ],
    )
    return _pallas_call(
        functools.partial(_agg_body, nheads, C),
        grid_spec=grid_spec,
        out_shape=[
            jax.ShapeDtypeStruct((NPAD, wd), jnp.float32),
            jax.ShapeDtypeStruct((NPAD, 16), jnp.float32),
        ],
        compiler_params=pltpu.CompilerParams(
            dimension_semantics=("arbitrary",)),
    )(blkmap, msg, a_mat_s, atd, dl3, mk3)


def _norm_body(nheads, C, mean_heads, out_ref, den_ref, b_ref, act_ref):
    den = den_ref[...]
    parts = []
    for h in range(nheads):
        s = den[:, h:h + 1]
        s_safe = jnp.where(s > 0, s, 1.0)
        parts.append(out_ref[:, h * C:(h + 1) * C] / s_safe)
    v = jnp.concatenate(parts, axis=1) if nheads > 1 else parts[0]
    if mean_heads and nheads > 1:
        v = sum(parts) / nheads
    act_ref[...] = jnp.maximum(v + b_ref[...], 0.0)


def _normalize(out, den, bias, nheads, C, mean_heads=False):
    wd = C if (mean_heads or nheads == 1) else nheads * C
    win = nheads * C
    return _pallas_call(
        functools.partial(_norm_body, nheads, C, mean_heads),
        grid=(NPAD // NB,),
        in_specs=[
            pl.BlockSpec((NB, win), lambda j: (j, 0)),
            pl.BlockSpec((NB, 16), lambda j: (j, 0)),
            pl.BlockSpec((1, wd), lambda j: (0, 0)),
        ],
        out_specs=pl.BlockSpec((NB, wd), lambda j: (j, 0)),
        out_shape=jax.ShapeDtypeStruct((NPAD, wd), jnp.float32),
    )(out, den, bias.reshape(1, wd))


def _pool_body(h_ref, b_ref, psum_ref, pcnt_ref):
    j = pl.program_id(0)
    bt = b_ref[0, 0, :]
    oh_t = (lax.broadcasted_iota(jnp.int32, (NG, NB), 0) == bt[None, :]
            ).astype(jnp.float32)
    ps = jnp.dot(oh_t, h_ref[...], preferred_element_type=jnp.float32)
    pc = jnp.dot(oh_t, jnp.ones((NB, 16), jnp.float32),
                 preferred_element_type=jnp.float32)

    @pl.when(j == 0)
    def _():
        psum_ref[...] = ps
        pcnt_ref[...] = pc

    @pl.when(j != 0)
    def _():
        psum_ref[...] = psum_ref[...] + ps
        pcnt_ref[...] = pcnt_ref[...] + pc


def _pool(h, batch_pad):
    b3 = batch_pad.reshape(NBLK, 1, NB)
    return _pallas_call(
        _pool_body,
        grid=(NBLK,),
        in_specs=[
            pl.BlockSpec((NB, HID), lambda j: (j, 0)),
            pl.BlockSpec((1, 1, NB), lambda j: (j, 0, 0)),
        ],
        out_specs=[
            pl.BlockSpec((NG, HID), lambda j: (0, 0)),
            pl.BlockSpec((NG, 16), lambda j: (0, 0)),
        ],
        out_shape=[
            jax.ShapeDtypeStruct((NG, HID), jnp.float32),
            jax.ShapeDtypeStruct((NG, 16), jnp.float32),
        ],
    )(h, b3)


def _heads_kernel(psum_ref, pcnt_ref, Wa1_ref, ba1_ref, Wa2_ref, ba2_ref,
                  Wc1_ref, bc1_ref, Wc2_ref, bc2_ref,
                  action_ref, value_ref):
    j = pl.program_id(0)
    cnt = jnp.maximum(pcnt_ref[:, 0:1], 1.0)
    pooled = psum_ref[...] / cnt
    ha = jax.nn.relu(pooled @ Wa1_ref[...] + ba1_ref[...][None, :])
    action_ref[...] = jnp.tanh(ha @ Wa2_ref[...] + ba2_ref[...])

    @pl.when(j == 0)
    def _():
        hc = jax.nn.relu(pooled @ Wc1_ref[...] + bc1_ref[...][None, :])
        value_ref[...] = hc @ Wc2_ref[...] + bc2_ref[...][None, :]


def _mlp_heads(psum, pcnt, Wa1, ba1, Wa2, ba2, Wc1, bc1, Wc2, bc2):
    BLK = 16000
    grid = (OUT_EDGES // BLK,)
    action, value = _pallas_call(
        _heads_kernel,
        grid=grid,
        in_specs=[
            pl.BlockSpec((NG, HID), lambda j: (0, 0)),
            pl.BlockSpec((NG, 16), lambda j: (0, 0)),
            pl.BlockSpec((HID, HID), lambda j: (0, 0)),
            pl.BlockSpec((HID,), lambda j: (0,)),
            pl.BlockSpec((HID, BLK), lambda j: (0, j)),
            pl.BlockSpec((1, BLK), lambda j: (0, j)),
            pl.BlockSpec((HID, HID), lambda j: (0, 0)),
            pl.BlockSpec((HID,), lambda j: (0,)),
            pl.BlockSpec((HID, 1), lambda j: (0, 0)),
            pl.BlockSpec((1,), lambda j: (0,)),
        ],
        out_specs=[
            pl.BlockSpec((NG, BLK), lambda j: (0, j)),
            pl.BlockSpec((NG, 1), lambda j: (0, 0)),
        ],
        out_shape=[
            jax.ShapeDtypeStruct((NG, OUT_EDGES), jnp.float32),
            jax.ShapeDtypeStruct((NG, 1), jnp.float32),
        ],
    )(psum, pcnt, Wa1, ba1, Wa2, ba2.reshape(1, -1), Wc1, bc1, Wc2, bc2)
    return action, value


def _gat_layer(act, W, a_s, a_d, b, nheads, C, idxs, mean_heads=False):
    src_pad, dst_pad, dstloc, maskf, blkmap = idxs
    h, atd = _dense(act, W, a_d, nheads, C)
    kw = 64 if nheads * C > 512 else 128
    msg = _sc_gather_rows(h, src_pad, nheads * C // 2, kw)
    out, den = _aggregate(msg, a_s, atd, dstloc, maskf, blkmap, nheads, C)
    return _normalize(out, den, b, nheads, C, mean_heads=mean_heads)


def kernel(x, edge_index, batch, W1, a1s, a1d, b1, W2, a2s, a2d, b2,
           W3, a3s, a3d, b3, Wa1, ba1, Wa2, ba2, Wc1, bc1, Wc2, bc2):
    idxs = _preprocess(edge_index)
    x_pad = jnp.pad(x, ((0, NPAD - NN), (0, 0)))
    batch_pad = jnp.pad(batch.astype(jnp.int32), (0, NPAD - NN),
                        constant_values=NG)
    h = _gat_layer(x_pad, W1, a1s, a1d, b1, 4, HID, idxs)
    h = _gat_layer(h, W2, a2s, a2d, b2, 4, HID, idxs)
    h = _gat_layer(h, W3, a3s, a3d, b3, 1, HID, idxs, mean_heads=True)
    psum, pcnt = _pool(h, batch_pad)
    action, value = _mlp_heads(psum, pcnt, Wa1, ba1, Wa2, ba2,
                               Wc1, bc1, Wc2, bc2)
    return (action, value)
